# baseline (speedup 1.0000x reference)
import functools

import jax
import jax.numpy as jnp
from jax import lax
from jax.experimental import pallas as pl
from jax.experimental.pallas import tpu as pltpu
from jax.experimental.pallas import tpu_sc as plsc

D_MODEL = 1024
D_FF = 4096
E = 8
T = 8192

TB = 1024
B = 128
MAX_BLOCKS = T // B + E - 1
PADDED = MAX_BLOCKS * B

NC, NS = 2, 16
NW = NC * NS
TPW = T // NW
CHUNK = 64
NCHUNK = TPW // CHUNK


def _router_body(x_ref, rw_ref, w16_ref, idx_ref, rank_ref, counts_ref,
                 carry_ref):
    g = pl.program_id(0)

    @pl.when(g == 0)
    def _():
        carry_ref[...] = jnp.zeros_like(carry_ref)

    x = x_ref[...]
    logits = lax.dot_general(x, rw_ref[...],
                             (((1,), (1,)), ((), ())),
                             preferred_element_type=jnp.float32)
    m = jnp.max(logits, axis=1, keepdims=True)
    s = jnp.sum(jnp.exp(logits - m), axis=1, keepdims=True)
    w = 1.0 / s
    w16_ref[...] = jnp.broadcast_to(w, (TB, 128))

    e_iota = lax.broadcasted_iota(jnp.int32, (TB, E), 1)
    idx = jnp.min(jnp.where(logits == m, e_iota, E), axis=1)
    idx_ref[...] = idx

    oh = (lax.broadcasted_iota(jnp.int32, (E, TB), 0)
          == idx[None, :]).astype(jnp.float32)
    tl = (lax.broadcasted_iota(jnp.int32, (TB, TB), 0)
          < lax.broadcasted_iota(jnp.int32, (TB, TB), 1)).astype(jnp.float32)
    within = lax.dot_general(oh, tl, (((1,), (0,)), ((), ())),
                             preferred_element_type=jnp.float32)
    carry = carry_ref[...][:, 0:1]
    rank = jnp.sum(oh * (within + carry), axis=0)
    rank_ref[...] = rank.astype(jnp.int32)

    new_carry = carry + jnp.sum(oh, axis=1, keepdims=True)
    carry_ref[...] = jnp.broadcast_to(new_carry, (E, 128))
    counts_ref[...] = jnp.broadcast_to(new_carry, (E, 128)).astype(jnp.int32)


def _run_router(x, router_w):
    return pl.pallas_call(
        _router_body,
        grid=(T // TB,),
        in_specs=[
            pl.BlockSpec((TB, D_MODEL), lambda g: (g, 0)),
            pl.BlockSpec((E, D_MODEL), lambda g: (0, 0)),
        ],
        out_specs=[
            pl.BlockSpec((TB, 128), lambda g: (g, 0)),
            pl.BlockSpec((TB,), lambda g: (g,)),
            pl.BlockSpec((TB,), lambda g: (g,)),
            pl.BlockSpec((E, 128), lambda g: (0, 0)),
        ],
        out_shape=[
            jax.ShapeDtypeStruct((T, 128), jnp.float32),
            jax.ShapeDtypeStruct((T,), jnp.int32),
            jax.ShapeDtypeStruct((T,), jnp.int32),
            jax.ShapeDtypeStruct((E, 128), jnp.int32),
        ],
        scratch_shapes=[pltpu.VMEM((E, 128), jnp.float32)],
    )(x, router_w)


def _layout_body(counts_ref, idx_ref, rank_ref, dst_ref, be_ref):
    counts = counts_ref[...][:, 0:1]
    nb = lax.shift_right_logical(counts + (B - 1), B.bit_length() - 1)
    nb_f = nb.astype(jnp.float32)
    tli = (lax.broadcasted_iota(jnp.int32, (E, E), 0)
           <= lax.broadcasted_iota(jnp.int32, (E, E), 1)).astype(jnp.float32)
    end_b = lax.dot_general(nb_f[:, 0][None, :], tli,
                            (((1,), (0,)), ((), ())),
                            preferred_element_type=jnp.float32)
    end_b = end_b.astype(jnp.int32)
    start_b = end_b - nb[:, 0][None, :]
    region_start = start_b * B

    idx = idx_ref[...]
    oh = (lax.broadcasted_iota(jnp.int32, (E, T), 0) == idx[None, :])
    dst = jnp.sum(jnp.where(oh, region_start.reshape(E, 1), 0), axis=0)
    dst_ref[...] = dst + rank_ref[...]

    g_iota = lax.broadcasted_iota(jnp.int32, (E, 128), 1)
    be = jnp.sum((g_iota >= end_b.reshape(E, 1)).astype(jnp.int32), axis=0)
    be_ref[...] = jnp.minimum(be, E - 1)


def _run_layout(counts, idx, rank):
    return pl.pallas_call(
        _layout_body,
        in_specs=[
            pl.BlockSpec((E, 128), lambda: (0, 0)),
            pl.BlockSpec((T,), lambda: (0,)),
            pl.BlockSpec((T,), lambda: (0,)),
        ],
        out_specs=[
            pl.BlockSpec((T,), lambda: (0,)),
            pl.BlockSpec((128,), lambda: (0,)),
        ],
        out_shape=[
            jax.ShapeDtypeStruct((T,), jnp.int32),
            jax.ShapeDtypeStruct((128,), jnp.int32),
        ],
    )(counts, idx, rank)


def _sc_scatter(x, w16, dst2d):
    mesh = plsc.VectorSubcoreMesh(core_axis_name="c", subcore_axis_name="s")

    @functools.partial(
        pl.kernel,
        mesh=mesh,
        out_type=(
            jax.ShapeDtypeStruct((PADDED, D_MODEL), jnp.float32),
            jax.ShapeDtypeStruct((PADDED, 128), jnp.float32),
        ),
        scratch_types=[
            pltpu.VMEM((NCHUNK, CHUNK), jnp.int32),
            pltpu.VMEM((CHUNK, D_MODEL), jnp.float32),
            pltpu.VMEM((CHUNK, 128), jnp.float32),
            pltpu.SemaphoreType.DMA,
        ],
    )
    def k(x_hbm, w16_hbm, dst_hbm, xs_hbm, ws_hbm, idx_v, rows_v, wrows_v,
          sem):
        wid = lax.axis_index("s") * NC + lax.axis_index("c")
        pltpu.sync_copy(dst_hbm.at[pl.ds(wid * NCHUNK, NCHUNK)], idx_v)
        base = wid * TPW
        for c in range(NCHUNK):
            pltpu.sync_copy(x_hbm.at[pl.ds(base + c * CHUNK, CHUNK)], rows_v)
            pltpu.async_copy(rows_v, xs_hbm.at[idx_v.at[c]], sem).wait()
            pltpu.sync_copy(w16_hbm.at[pl.ds(base + c * CHUNK, CHUNK)],
                            wrows_v)
            pltpu.async_copy(wrows_v, ws_hbm.at[idx_v.at[c]], sem).wait()

    return k(x, w16, dst2d)


def _ffn_body(be_ref, x_ref, w1_hbm, w2_hbm, ws_ref, y_ref,
              w1_v, w2_v, slot_ref, sem1, sem2):
    g = pl.program_id(0)
    e = be_ref[g]
    switched = (g == 0) | (e != be_ref[jnp.maximum(g - 1, 0)])

    @pl.when(g == 0)
    def _():
        slot_ref[0] = 0
        pltpu.make_async_copy(w1_hbm.at[e], w1_v.at[0], sem1).start()

    @pl.when((g > 0) & switched)
    def _():
        slot_ref[0] = 1 - slot_ref[0]

    s = slot_ref[0]

    @pl.when(switched)
    def _():
        pltpu.make_async_copy(w1_hbm.at[0], w1_v.at[s], sem1).wait()
        pltpu.make_async_copy(w2_hbm.at[e], w2_v, sem2).start()
        j = lax.while_loop(
            lambda j: (j < MAX_BLOCKS) & (be_ref[jnp.minimum(j, MAX_BLOCKS - 1)] == e),
            lambda j: j + 1, g + 1)

        @pl.when(j < MAX_BLOCKS)
        def _():
            e_nd = be_ref[jnp.minimum(j, MAX_BLOCKS - 1)]
            pltpu.make_async_copy(w1_hbm.at[e_nd], w1_v.at[1 - s],
                                  sem1).start()

    x = x_ref[...]
    h = lax.dot_general(x, w1_v[s], (((1,), (1,)), ((), ())),
                        preferred_element_type=jnp.float32)
    h = h * (1.0 / (1.0 + jnp.exp(-h)))

    @pl.when(switched)
    def _():
        pltpu.make_async_copy(w2_hbm.at[0], w2_v, sem2).wait()

    y = lax.dot_general(h, w2_v[...], (((1,), (1,)), ((), ())),
                        preferred_element_type=jnp.float32)
    y_ref[...] = y * ws_ref[...][:, 0:1]


def _run_ffn(block_expert, x_sorted, w1, w2, w_sorted):
    grid_spec = pltpu.PrefetchScalarGridSpec(
        num_scalar_prefetch=1,
        grid=(MAX_BLOCKS,),
        in_specs=[
            pl.BlockSpec((B, D_MODEL), lambda g, be: (g, 0)),
            pl.BlockSpec(memory_space=pl.ANY),
            pl.BlockSpec(memory_space=pl.ANY),
            pl.BlockSpec((B, 128), lambda g, be: (g, 0)),
        ],
        out_specs=pl.BlockSpec((B, D_MODEL), lambda g, be: (g, 0)),
        scratch_shapes=[
            pltpu.VMEM((2, D_FF, D_MODEL), jnp.float32),
            pltpu.VMEM((D_MODEL, D_FF), jnp.float32),
            pltpu.SMEM((1,), jnp.int32),
            pltpu.SemaphoreType.DMA,
            pltpu.SemaphoreType.DMA,
        ],
    )
    return pl.pallas_call(
        _ffn_body,
        grid_spec=grid_spec,
        out_shape=jax.ShapeDtypeStruct((PADDED, D_MODEL), jnp.float32),
        compiler_params=pltpu.CompilerParams(
            vmem_limit_bytes=100 * 1024 * 1024),
    )(block_expert, x_sorted, w1, w2, w_sorted)


def _sc_gather(y_sorted, dst2d):
    mesh = plsc.VectorSubcoreMesh(core_axis_name="c", subcore_axis_name="s")

    @functools.partial(
        pl.kernel,
        mesh=mesh,
        out_type=jax.ShapeDtypeStruct((T, D_MODEL), jnp.float32),
        scratch_types=[
            pltpu.VMEM((NCHUNK, CHUNK), jnp.int32),
            pltpu.VMEM((CHUNK, D_MODEL), jnp.float32),
            pltpu.SemaphoreType.DMA,
        ],
    )
    def k(ys_hbm, dst_hbm, out_hbm, idx_v, rows_v, sem):
        wid = lax.axis_index("s") * NC + lax.axis_index("c")
        pltpu.sync_copy(dst_hbm.at[pl.ds(wid * NCHUNK, NCHUNK)], idx_v)
        base = wid * TPW
        for c in range(NCHUNK):
            pltpu.async_copy(ys_hbm.at[idx_v.at[c]], rows_v, sem).wait()
            pltpu.sync_copy(rows_v, out_hbm.at[pl.ds(base + c * CHUNK, CHUNK)])

    return k(y_sorted, dst2d)


def kernel(x, router_w, w1, w2):
    w16, idx, rank, counts = _run_router(x, router_w)
    dst, block_expert = _run_layout(counts, idx, rank)
    dst2d = dst.reshape(T // CHUNK, CHUNK)
    x_sorted, w_sorted = _sc_scatter(x, w16, dst2d)
    y_sorted = _run_ffn(block_expert, x_sorted, w1, w2, w_sorted)
    return _sc_gather(y_sorted, dst2d)


# speedup vs baseline: 1.6749x; 1.6749x over previous
import functools

import jax
import jax.numpy as jnp
from jax import lax
from jax.experimental import pallas as pl
from jax.experimental.pallas import tpu as pltpu
from jax.experimental.pallas import tpu_sc as plsc

D_MODEL = 1024
D_FF = 4096
E = 8
T = 8192

TB = 1024
B = 256
MAX_BLOCKS = T // B + E - 1
PADDED = MAX_BLOCKS * B

NC, NS = 2, 16
NW = NC * NS
TPW = T // NW
CHUNK = 64
NCHUNK = TPW // CHUNK
W2C = 4


def _router_body(x_ref, rw_ref, w16_ref, idx_ref, rank_ref, counts_ref,
                 carry_ref):
    g = pl.program_id(0)

    @pl.when(g == 0)
    def _():
        carry_ref[...] = jnp.zeros_like(carry_ref)

    x = x_ref[...]
    logits = lax.dot_general(x, rw_ref[...],
                             (((1,), (1,)), ((), ())),
                             preferred_element_type=jnp.float32)
    m = jnp.max(logits, axis=1, keepdims=True)
    s = jnp.sum(jnp.exp(logits - m), axis=1, keepdims=True)
    w = 1.0 / s
    w16_ref[...] = jnp.broadcast_to(w, (TB, 128))

    e_iota = lax.broadcasted_iota(jnp.int32, (TB, E), 1)
    idx = jnp.min(jnp.where(logits == m, e_iota, E), axis=1)
    idx_ref[...] = idx

    oh = (lax.broadcasted_iota(jnp.int32, (E, TB), 0)
          == idx[None, :]).astype(jnp.float32)
    tl = (lax.broadcasted_iota(jnp.int32, (TB, TB), 0)
          < lax.broadcasted_iota(jnp.int32, (TB, TB), 1)).astype(jnp.float32)
    within = lax.dot_general(oh, tl, (((1,), (0,)), ((), ())),
                             preferred_element_type=jnp.float32)
    carry = carry_ref[...][:, 0:1]
    rank = jnp.sum(oh * (within + carry), axis=0)
    rank_ref[...] = rank.astype(jnp.int32)

    new_carry = carry + jnp.sum(oh, axis=1, keepdims=True)
    carry_ref[...] = jnp.broadcast_to(new_carry, (E, 128))
    counts_ref[...] = jnp.broadcast_to(new_carry, (E, 128)).astype(jnp.int32)


def _run_router(x, router_w):
    return pl.pallas_call(
        _router_body,
        grid=(T // TB,),
        in_specs=[
            pl.BlockSpec((TB, D_MODEL), lambda g: (g, 0)),
            pl.BlockSpec((E, D_MODEL), lambda g: (0, 0)),
        ],
        out_specs=[
            pl.BlockSpec((TB, 128), lambda g: (g, 0)),
            pl.BlockSpec((TB,), lambda g: (g,)),
            pl.BlockSpec((TB,), lambda g: (g,)),
            pl.BlockSpec((E, 128), lambda g: (0, 0)),
        ],
        out_shape=[
            jax.ShapeDtypeStruct((T, 128), jnp.float32),
            jax.ShapeDtypeStruct((T,), jnp.int32),
            jax.ShapeDtypeStruct((T,), jnp.int32),
            jax.ShapeDtypeStruct((E, 128), jnp.int32),
        ],
        scratch_shapes=[pltpu.VMEM((E, 128), jnp.float32)],
    )(x, router_w)


def _layout_body(counts_ref, idx_ref, rank_ref, dst_ref, be_ref):
    counts = counts_ref[...][:, 0:1]
    nb = lax.shift_right_logical(counts + (B - 1), B.bit_length() - 1)
    nb_f = nb.astype(jnp.float32)
    tli = (lax.broadcasted_iota(jnp.int32, (E, E), 0)
           <= lax.broadcasted_iota(jnp.int32, (E, E), 1)).astype(jnp.float32)
    end_b = lax.dot_general(nb_f[:, 0][None, :], tli,
                            (((1,), (0,)), ((), ())),
                            preferred_element_type=jnp.float32)
    end_b = end_b.astype(jnp.int32)
    start_b = end_b - nb[:, 0][None, :]
    region_start = start_b * B

    idx = idx_ref[...]
    oh = (lax.broadcasted_iota(jnp.int32, (E, T), 0) == idx[None, :])
    dst = jnp.sum(jnp.where(oh, region_start.reshape(E, 1), 0), axis=0)
    dst_ref[...] = dst + rank_ref[...]

    g_iota = lax.broadcasted_iota(jnp.int32, (E, 128), 1)
    be = jnp.sum((g_iota >= end_b.reshape(E, 1)).astype(jnp.int32), axis=0)
    be_ref[...] = jnp.minimum(be, E - 1)


def _run_layout(counts, idx, rank):
    return pl.pallas_call(
        _layout_body,
        in_specs=[
            pl.BlockSpec((E, 128), lambda: (0, 0)),
            pl.BlockSpec((T,), lambda: (0,)),
            pl.BlockSpec((T,), lambda: (0,)),
        ],
        out_specs=[
            pl.BlockSpec((T,), lambda: (0,)),
            pl.BlockSpec((128,), lambda: (0,)),
        ],
        out_shape=[
            jax.ShapeDtypeStruct((T,), jnp.int32),
            jax.ShapeDtypeStruct((128,), jnp.int32),
        ],
    )(counts, idx, rank)


def _sc_scatter(x, w16, dst2d):
    mesh = plsc.VectorSubcoreMesh(core_axis_name="c", subcore_axis_name="s")

    @functools.partial(
        pl.kernel,
        mesh=mesh,
        out_type=(
            jax.ShapeDtypeStruct((PADDED, D_MODEL), jnp.float32),
            jax.ShapeDtypeStruct((PADDED, 128), jnp.float32),
        ),
        scratch_types=[
            pltpu.VMEM((NCHUNK, CHUNK), jnp.int32),
            pltpu.VMEM((CHUNK, D_MODEL), jnp.float32),
            pltpu.VMEM((CHUNK, 128), jnp.float32),
            pltpu.SemaphoreType.DMA,
        ],
    )
    def k(x_hbm, w16_hbm, dst_hbm, xs_hbm, ws_hbm, idx_v, rows_v, wrows_v,
          sem):
        wid = lax.axis_index("s") * NC + lax.axis_index("c")
        pltpu.sync_copy(dst_hbm.at[pl.ds(wid * NCHUNK, NCHUNK)], idx_v)
        base = wid * TPW
        for c in range(NCHUNK):
            pltpu.sync_copy(x_hbm.at[pl.ds(base + c * CHUNK, CHUNK)], rows_v)
            pltpu.async_copy(rows_v, xs_hbm.at[idx_v.at[c]], sem).wait()
            pltpu.sync_copy(w16_hbm.at[pl.ds(base + c * CHUNK, CHUNK)],
                            wrows_v)
            pltpu.async_copy(wrows_v, ws_hbm.at[idx_v.at[c]], sem).wait()

    return k(x, w16, dst2d)


def _ffn_body(be_ref, x_ref, w1_hbm, w2_hbm, ws_ref, y_ref,
              w1_v, w2_v, slot_ref, sem1, sem2):
    g = pl.program_id(0)
    e = be_ref[g]
    switched = (g == 0) | (e != be_ref[jnp.maximum(g - 1, 0)])

    @pl.when(g == 0)
    def _():
        slot_ref[0] = 0
        pltpu.make_async_copy(w1_hbm.at[e], w1_v.at[0], sem1).start()

    @pl.when((g > 0) & switched)
    def _():
        slot_ref[0] = 1 - slot_ref[0]

    s = slot_ref[0]

    @pl.when(switched)
    def _():
        pltpu.make_async_copy(w1_hbm.at[0], w1_v.at[s], sem1).wait()
        for k in range(W2C):
            pltpu.make_async_copy(
                w2_hbm.at[e, :, pl.ds(k * (D_FF // W2C), D_FF // W2C)],
                w2_v.at[k], sem2).start()
        j = lax.while_loop(
            lambda j: (j < MAX_BLOCKS) & (be_ref[jnp.minimum(j, MAX_BLOCKS - 1)] == e),
            lambda j: j + 1, g + 1)

        @pl.when(j < MAX_BLOCKS)
        def _():
            e_nd = be_ref[jnp.minimum(j, MAX_BLOCKS - 1)]
            pltpu.make_async_copy(w1_hbm.at[e_nd], w1_v.at[1 - s],
                                  sem1).start()

    x = x_ref[...]
    h = lax.dot_general(x, w1_v[s], (((1,), (1,)), ((), ())),
                        preferred_element_type=jnp.float32)
    h = h * (1.0 / (1.0 + jnp.exp(-h)))

    fc = D_FF // W2C
    y = jnp.zeros((B, D_MODEL), jnp.float32)
    for k in range(W2C):
        @pl.when(switched)
        def _():
            pltpu.make_async_copy(w2_hbm.at[0, :, pl.ds(0, fc)],
                                  w2_v.at[k], sem2).wait()

        y = y + lax.dot_general(h[:, k * fc:(k + 1) * fc], w2_v[k],
                                (((1,), (1,)), ((), ())),
                                preferred_element_type=jnp.float32)
    y_ref[...] = y * ws_ref[...][:, 0:1]


def _run_ffn(block_expert, x_sorted, w1, w2, w_sorted):
    grid_spec = pltpu.PrefetchScalarGridSpec(
        num_scalar_prefetch=1,
        grid=(MAX_BLOCKS,),
        in_specs=[
            pl.BlockSpec((B, D_MODEL), lambda g, be: (g, 0)),
            pl.BlockSpec(memory_space=pl.ANY),
            pl.BlockSpec(memory_space=pl.ANY),
            pl.BlockSpec((B, 128), lambda g, be: (g, 0)),
        ],
        out_specs=pl.BlockSpec((B, D_MODEL), lambda g, be: (g, 0)),
        scratch_shapes=[
            pltpu.VMEM((2, D_FF, D_MODEL), jnp.float32),
            pltpu.VMEM((W2C, D_MODEL, D_FF // W2C), jnp.float32),
            pltpu.SMEM((1,), jnp.int32),
            pltpu.SemaphoreType.DMA,
            pltpu.SemaphoreType.DMA,
        ],
    )
    return pl.pallas_call(
        _ffn_body,
        grid_spec=grid_spec,
        out_shape=jax.ShapeDtypeStruct((PADDED, D_MODEL), jnp.float32),
        compiler_params=pltpu.CompilerParams(
            vmem_limit_bytes=100 * 1024 * 1024),
    )(block_expert, x_sorted, w1, w2, w_sorted)


def _sc_gather(y_sorted, dst2d):
    mesh = plsc.VectorSubcoreMesh(core_axis_name="c", subcore_axis_name="s")

    @functools.partial(
        pl.kernel,
        mesh=mesh,
        out_type=jax.ShapeDtypeStruct((T, D_MODEL), jnp.float32),
        scratch_types=[
            pltpu.VMEM((NCHUNK, CHUNK), jnp.int32),
            pltpu.VMEM((CHUNK, D_MODEL), jnp.float32),
            pltpu.SemaphoreType.DMA,
        ],
    )
    def k(ys_hbm, dst_hbm, out_hbm, idx_v, rows_v, sem):
        wid = lax.axis_index("s") * NC + lax.axis_index("c")
        pltpu.sync_copy(dst_hbm.at[pl.ds(wid * NCHUNK, NCHUNK)], idx_v)
        base = wid * TPW
        for c in range(NCHUNK):
            pltpu.async_copy(ys_hbm.at[idx_v.at[c]], rows_v, sem).wait()
            pltpu.sync_copy(rows_v, out_hbm.at[pl.ds(base + c * CHUNK, CHUNK)])

    return k(y_sorted, dst2d)


def kernel(x, router_w, w1, w2):
    w16, idx, rank, counts = _run_router(x, router_w)
    dst, block_expert = _run_layout(counts, idx, rank)
    dst2d = dst.reshape(T // CHUNK, CHUNK)
    x_sorted, w_sorted = _sc_scatter(x, w16, dst2d)
    y_sorted = _run_ffn(block_expert, x_sorted, w1, w2, w_sorted)
    return _sc_gather(y_sorted, dst2d)


# speedup vs baseline: 1.7371x; 1.0372x over previous
import functools

import jax
import jax.numpy as jnp
from jax import lax
from jax.experimental import pallas as pl
from jax.experimental.pallas import tpu as pltpu
from jax.experimental.pallas import tpu_sc as plsc

D_MODEL = 1024
D_FF = 4096
E = 8
T = 8192

TB = 1024
B = 256
MAX_BLOCKS = T // B + E - 1
PADDED = MAX_BLOCKS * B

NC, NS = 2, 16
NW = NC * NS
TPW = T // NW
CHUNK = 64
NCHUNK = TPW // CHUNK
W2C = 4


def _router_body(x_ref, rw_ref, w16_ref, idx_ref, rank_ref, counts_ref,
                 carry_ref):
    g = pl.program_id(0)

    @pl.when(g == 0)
    def _():
        carry_ref[...] = jnp.zeros_like(carry_ref)

    x = x_ref[...]
    logits = lax.dot_general(x, rw_ref[...],
                             (((1,), (1,)), ((), ())),
                             preferred_element_type=jnp.float32)
    m = jnp.max(logits, axis=1, keepdims=True)
    s = jnp.sum(jnp.exp(logits - m), axis=1, keepdims=True)
    w = 1.0 / s
    w16_ref[...] = jnp.broadcast_to(w, (TB, 128))

    e_iota = lax.broadcasted_iota(jnp.int32, (TB, E), 1)
    idx = jnp.min(jnp.where(logits == m, e_iota, E), axis=1)
    idx_ref[...] = idx

    oh = (lax.broadcasted_iota(jnp.int32, (E, TB), 0)
          == idx[None, :]).astype(jnp.float32)
    tl = (lax.broadcasted_iota(jnp.int32, (TB, TB), 0)
          < lax.broadcasted_iota(jnp.int32, (TB, TB), 1)).astype(jnp.float32)
    within = lax.dot_general(oh, tl, (((1,), (0,)), ((), ())),
                             preferred_element_type=jnp.float32)
    carry = carry_ref[...][:, 0:1]
    rank = jnp.sum(oh * (within + carry), axis=0)
    rank_ref[...] = rank.astype(jnp.int32)

    new_carry = carry + jnp.sum(oh, axis=1, keepdims=True)
    carry_ref[...] = jnp.broadcast_to(new_carry, (E, 128))
    counts_ref[...] = jnp.broadcast_to(new_carry, (E, 128)).astype(jnp.int32)


def _run_router(x, router_w):
    return pl.pallas_call(
        _router_body,
        grid=(T // TB,),
        in_specs=[
            pl.BlockSpec((TB, D_MODEL), lambda g: (g, 0)),
            pl.BlockSpec((E, D_MODEL), lambda g: (0, 0)),
        ],
        out_specs=[
            pl.BlockSpec((TB, 128), lambda g: (g, 0)),
            pl.BlockSpec((TB,), lambda g: (g,)),
            pl.BlockSpec((TB,), lambda g: (g,)),
            pl.BlockSpec((E, 128), lambda g: (0, 0)),
        ],
        out_shape=[
            jax.ShapeDtypeStruct((T, 128), jnp.float32),
            jax.ShapeDtypeStruct((T,), jnp.int32),
            jax.ShapeDtypeStruct((T,), jnp.int32),
            jax.ShapeDtypeStruct((E, 128), jnp.int32),
        ],
        scratch_shapes=[pltpu.VMEM((E, 128), jnp.float32)],
    )(x, router_w)


def _layout_body(counts_ref, idx_ref, rank_ref, dst_ref, be_ref):
    counts = counts_ref[...][:, 0:1]
    nb = lax.shift_right_logical(counts + (B - 1), B.bit_length() - 1)
    nb_f = nb.astype(jnp.float32)
    tli = (lax.broadcasted_iota(jnp.int32, (E, E), 0)
           <= lax.broadcasted_iota(jnp.int32, (E, E), 1)).astype(jnp.float32)
    end_b = lax.dot_general(nb_f[:, 0][None, :], tli,
                            (((1,), (0,)), ((), ())),
                            preferred_element_type=jnp.float32)
    end_b = end_b.astype(jnp.int32)
    start_b = end_b - nb[:, 0][None, :]
    region_start = start_b * B

    idx = idx_ref[...]
    oh = (lax.broadcasted_iota(jnp.int32, (E, T), 0) == idx[None, :])
    dst = jnp.sum(jnp.where(oh, region_start.reshape(E, 1), 0), axis=0)
    dst_ref[...] = dst + rank_ref[...]

    g_iota = lax.broadcasted_iota(jnp.int32, (E, 128), 1)
    be = jnp.sum((g_iota >= end_b.reshape(E, 1)).astype(jnp.int32), axis=0)
    be_ref[...] = jnp.minimum(be, E - 1)


def _run_layout(counts, idx, rank):
    return pl.pallas_call(
        _layout_body,
        in_specs=[
            pl.BlockSpec((E, 128), lambda: (0, 0)),
            pl.BlockSpec((T,), lambda: (0,)),
            pl.BlockSpec((T,), lambda: (0,)),
        ],
        out_specs=[
            pl.BlockSpec((T,), lambda: (0,)),
            pl.BlockSpec((128,), lambda: (0,)),
        ],
        out_shape=[
            jax.ShapeDtypeStruct((T,), jnp.int32),
            jax.ShapeDtypeStruct((128,), jnp.int32),
        ],
    )(counts, idx, rank)


def _sc_scatter(x, w16, dst2d):
    mesh = plsc.VectorSubcoreMesh(core_axis_name="c", subcore_axis_name="s")

    @functools.partial(
        pl.kernel,
        mesh=mesh,
        out_type=(
            jax.ShapeDtypeStruct((PADDED, D_MODEL), jnp.float32),
            jax.ShapeDtypeStruct((PADDED, 128), jnp.float32),
        ),
        scratch_types=[
            pltpu.VMEM((NCHUNK, CHUNK), jnp.int32),
            pltpu.VMEM((CHUNK, D_MODEL), jnp.float32),
            pltpu.VMEM((CHUNK, 128), jnp.float32),
            pltpu.SemaphoreType.DMA,
        ],
    )
    def k(x_hbm, w16_hbm, dst_hbm, xs_hbm, ws_hbm, idx_v, rows_v, wrows_v,
          sem):
        wid = lax.axis_index("s") * NC + lax.axis_index("c")
        pltpu.sync_copy(dst_hbm.at[pl.ds(wid * NCHUNK, NCHUNK)], idx_v)
        base = wid * TPW
        for c in range(NCHUNK):
            pltpu.sync_copy(x_hbm.at[pl.ds(base + c * CHUNK, CHUNK)], rows_v)
            pltpu.async_copy(rows_v, xs_hbm.at[idx_v.at[c]], sem).wait()
            pltpu.sync_copy(w16_hbm.at[pl.ds(base + c * CHUNK, CHUNK)],
                            wrows_v)
            pltpu.async_copy(wrows_v, ws_hbm.at[idx_v.at[c]], sem).wait()

    return k(x, w16, dst2d)


def _ffn_body(be_ref, x_ref, w1_hbm, w2_hbm, ws_ref, y_ref,
              w1_v, w2_v, slot_ref, sem1, sem2):
    g = pl.program_id(0)
    e = be_ref[g]
    switched = (g == 0) | (e != be_ref[jnp.maximum(g - 1, 0)])

    @pl.when(g == 0)
    def _():
        slot_ref[0] = 0
        pltpu.make_async_copy(w1_hbm.at[e], w1_v.at[0], sem1).start()
        pltpu.make_async_copy(w2_hbm.at[e], w2_v, sem2).start()

    @pl.when((g > 0) & switched)
    def _():
        slot_ref[0] = 1 - slot_ref[0]

    s = slot_ref[0]

    @pl.when(switched)
    def _():
        pltpu.make_async_copy(w1_hbm.at[0], w1_v.at[s], sem1).wait()
        j = lax.while_loop(
            lambda j: (j < MAX_BLOCKS) & (be_ref[jnp.minimum(j, MAX_BLOCKS - 1)] == e),
            lambda j: j + 1, g + 1)

        @pl.when(j < MAX_BLOCKS)
        def _():
            e_nd = be_ref[jnp.minimum(j, MAX_BLOCKS - 1)]
            pltpu.make_async_copy(w1_hbm.at[e_nd], w1_v.at[1 - s],
                                  sem1).start()

    x = x_ref[...]
    h = lax.dot_general(x, w1_v[s], (((1,), (1,)), ((), ())),
                        preferred_element_type=jnp.float32)
    h = h * (1.0 / (1.0 + jnp.exp(-h)))

    @pl.when(switched)
    def _():
        pltpu.make_async_copy(w2_hbm.at[0], w2_v, sem2).wait()

    y = lax.dot_general(h, w2_v[...], (((1,), (1,)), ((), ())),
                        preferred_element_type=jnp.float32)
    y_ref[...] = y * ws_ref[...][:, 0:1]

    e_next2 = be_ref[jnp.minimum(g + 1, MAX_BLOCKS - 1)]

    @pl.when((g + 1 < MAX_BLOCKS) & (e_next2 != e))
    def _():
        pltpu.make_async_copy(w2_hbm.at[e_next2], w2_v, sem2).start()


def _run_ffn(block_expert, x_sorted, w1, w2, w_sorted):
    grid_spec = pltpu.PrefetchScalarGridSpec(
        num_scalar_prefetch=1,
        grid=(MAX_BLOCKS,),
        in_specs=[
            pl.BlockSpec((B, D_MODEL), lambda g, be: (g, 0)),
            pl.BlockSpec(memory_space=pl.ANY),
            pl.BlockSpec(memory_space=pl.ANY),
            pl.BlockSpec((B, 128), lambda g, be: (g, 0)),
        ],
        out_specs=pl.BlockSpec((B, D_MODEL), lambda g, be: (g, 0)),
        scratch_shapes=[
            pltpu.VMEM((2, D_FF, D_MODEL), jnp.float32),
            pltpu.VMEM((D_MODEL, D_FF), jnp.float32),
            pltpu.SMEM((1,), jnp.int32),
            pltpu.SemaphoreType.DMA,
            pltpu.SemaphoreType.DMA,
        ],
    )
    return pl.pallas_call(
        _ffn_body,
        grid_spec=grid_spec,
        out_shape=jax.ShapeDtypeStruct((PADDED, D_MODEL), jnp.float32),
        compiler_params=pltpu.CompilerParams(
            vmem_limit_bytes=100 * 1024 * 1024),
    )(block_expert, x_sorted, w1, w2, w_sorted)


def _sc_gather(y_sorted, dst2d):
    mesh = plsc.VectorSubcoreMesh(core_axis_name="c", subcore_axis_name="s")

    @functools.partial(
        pl.kernel,
        mesh=mesh,
        out_type=jax.ShapeDtypeStruct((T, D_MODEL), jnp.float32),
        scratch_types=[
            pltpu.VMEM((NCHUNK, CHUNK), jnp.int32),
            pltpu.VMEM((CHUNK, D_MODEL), jnp.float32),
            pltpu.SemaphoreType.DMA,
        ],
    )
    def k(ys_hbm, dst_hbm, out_hbm, idx_v, rows_v, sem):
        wid = lax.axis_index("s") * NC + lax.axis_index("c")
        pltpu.sync_copy(dst_hbm.at[pl.ds(wid * NCHUNK, NCHUNK)], idx_v)
        base = wid * TPW
        for c in range(NCHUNK):
            pltpu.async_copy(ys_hbm.at[idx_v.at[c]], rows_v, sem).wait()
            pltpu.sync_copy(rows_v, out_hbm.at[pl.ds(base + c * CHUNK, CHUNK)])

    return k(y_sorted, dst2d)


def kernel(x, router_w, w1, w2):
    w16, idx, rank, counts = _run_router(x, router_w)
    dst, block_expert = _run_layout(counts, idx, rank)
    dst2d = dst.reshape(T // CHUNK, CHUNK)
    x_sorted, w_sorted = _sc_scatter(x, w16, dst2d)
    y_sorted = _run_ffn(block_expert, x_sorted, w1, w2, w_sorted)
    return _sc_gather(y_sorted, dst2d)
